# baseline (speedup 1.0000x reference)
import functools

import jax
import jax.numpy as jnp
from jax import lax
from jax.experimental import pallas as pl
from jax.experimental.pallas import tpu as pltpu
from jax.experimental.pallas import tpu_sc as plsc

NREAL = 10000
E = 160000
CLIP = 128
H = 2
HID = 256
PRE = 2
MB = 128
OUT2 = PRE * MB
B = 16

NC = 2
NS = 16
NW = NC * NS
LN = 16
NP = 10240
R = NP // NW
EP = 163840
CH = 2048
NCH = EP // CH
CAP = 6160
GK = 32


def _proj_body(x_ref, w_ref, wsdt_ref, xp_ref, asdt_ref):
    xb = x_ref[...]
    xp_ref[...] = jnp.dot(
        xb, w_ref[...], preferred_element_type=jnp.float32
    ).astype(jnp.bfloat16)
    asdt_ref[...] = lax.dot_general(
        wsdt_ref[...], xb, (((1,), (1,)), ((), ())),
        preferred_element_type=jnp.float32)


def _proj(xpad, W, wsdT):
    K = xpad.shape[1]
    HW = W.shape[1]
    blk = 1024
    return pl.pallas_call(
        _proj_body,
        grid=(NP // blk,),
        in_specs=[
            pl.BlockSpec((blk, K), lambda i: (i, 0)),
            pl.BlockSpec((K, HW), lambda i: (0, 0)),
            pl.BlockSpec((32, K), lambda i: (0, 0)),
        ],
        out_specs=[
            pl.BlockSpec((blk, HW), lambda i: (i, 0)),
            pl.BlockSpec((32, blk), lambda i: (0, i)),
        ],
        out_shape=[
            jax.ShapeDtypeStruct((NP, HW), jnp.bfloat16),
            jax.ShapeDtypeStruct((32, NP), jnp.float32),
        ],
    )(xpad, W, wsdT)


def _edge_body(ea_ref, wet_ref, out_ref):
    out_ref[...] = lax.dot_general(
        wet_ref[...], ea_ref[...], (((1,), (1,)), ((), ())),
        preferred_element_type=jnp.float32)


def _edge_logits(ea_pad, weT):
    blk = 2048
    return pl.pallas_call(
        _edge_body,
        grid=(EP // blk,),
        in_specs=[
            pl.BlockSpec((blk, CLIP), lambda i: (i, 0)),
            pl.BlockSpec((32, CLIP), lambda i: (0, 0)),
        ],
        out_specs=pl.BlockSpec((32, blk), lambda i: (0, i)),
        out_shape=jax.ShapeDtypeStruct((32, EP), jnp.float32),
    )(ea_pad, weT)


def _mlp_body(h_ref, g1_ref, gb1_ref, g2_ref, gb2_ref, g3_ref, ps_ref, out_ref):
    h = h_ref[...]
    gp1 = ps_ref[0, 0]
    gp2 = ps_ref[0, 1]
    gb3 = ps_ref[0, 2]
    t = jnp.dot(h, g1_ref[...], preferred_element_type=jnp.float32) + gb1_ref[...]
    t = jnp.where(t >= 0, t, gp1 * t)
    t = jnp.dot(t, g2_ref[...], preferred_element_type=jnp.float32) + gb2_ref[...]
    t = jnp.where(t >= 0, t, gp2 * t)
    out_ref[...] = jnp.dot(t, g3_ref[...], preferred_element_type=jnp.float32) + gb3


def _mlp(h, G1, gb1, gp1, G2, gb2, gp2, G3, gb3):
    g3p = jnp.pad(G3, ((0, 0), (0, 127)))
    ps = jnp.stack([gp1, gp2, gb3[0]]).reshape(1, 3)
    blk = 1024
    out = pl.pallas_call(
        _mlp_body,
        grid=(NP // blk,),
        in_specs=[
            pl.BlockSpec((blk, HID), lambda i: (i, 0)),
            pl.BlockSpec((HID, HID), lambda i: (0, 0)),
            pl.BlockSpec((1, HID), lambda i: (0, 0)),
            pl.BlockSpec((HID, HID), lambda i: (0, 0)),
            pl.BlockSpec((1, HID), lambda i: (0, 0)),
            pl.BlockSpec((HID, 128), lambda i: (0, 0)),
            pl.BlockSpec((1, 3), lambda i: (0, 0)),
        ],
        out_specs=pl.BlockSpec((blk, 128), lambda i: (i, 0)),
        out_shape=jax.ShapeDtypeStruct((NP, 128), jnp.float32),
    )(h, G1, gb1.reshape(1, HID), G2, gb2.reshape(1, HID), g3p, ps)
    return out


def _p1_body(g_ref, b_ref, o_ref):
    i = pl.program_id(0)

    @pl.when(i == 0)
    def _():
        o_ref[...] = jnp.full((B, 128), -1e30, jnp.float32)

    g_row = g_ref[0]
    bat = b_ref[0]
    M = bat == lax.broadcasted_iota(jnp.int32, (B, 1), 0)
    masked = jnp.where(M, g_row, -1e30)
    cur = jnp.max(masked, axis=1, keepdims=True)
    o_ref[...] = jnp.maximum(o_ref[...], jnp.broadcast_to(cur, (B, 128)))


def _p2_body(g_ref, b_ref, gm_ref, o_ref):
    i = pl.program_id(0)

    @pl.when(i == 0)
    def _():
        o_ref[...] = jnp.zeros((B, 128), jnp.float32)

    g_row = g_ref[0]
    bat = b_ref[0]
    M = bat == lax.broadcasted_iota(jnp.int32, (B, 1), 0)
    gmn = jnp.sum(jnp.where(M, gm_ref[:, 0:1], 0.0), axis=0, keepdims=True)
    ge = jnp.exp(g_row - gmn)
    cur = jnp.sum(M.astype(jnp.float32) * ge, axis=1, keepdims=True)
    o_ref[...] = o_ref[...] + jnp.broadcast_to(cur, (B, 128))


def _p3_body(g_ref, b_ref, h_ref, gm_ref, gd_ref, o_ref):
    i = pl.program_id(0)

    @pl.when(i == 0)
    def _():
        o_ref[...] = jnp.zeros((B, HID), jnp.float32)

    g_row = g_ref[0]
    bat = b_ref[0]
    M = bat == lax.broadcasted_iota(jnp.int32, (B, 1), 0)
    gmn = jnp.sum(jnp.where(M, gm_ref[:, 0:1], 0.0), axis=0, keepdims=True)
    gdn = jnp.sum(jnp.where(M, gd_ref[:, 0:1], 0.0), axis=0, keepdims=True)
    ge = jnp.exp(g_row - gmn)
    alpha = ge / (gdn + 1e-16)
    S = M.astype(jnp.float32) * alpha
    o_ref[...] = o_ref[...] + jnp.dot(S, h_ref[...],
                                      preferred_element_type=jnp.float32)


def _pool(g1, b3, h2r):
    nblk = 10
    gspec = pl.BlockSpec((1, 1, 1000), lambda i: (i, 0, 0))
    fullspec = lambda shp: pl.BlockSpec(shp, lambda i: (0, 0))
    gmax = pl.pallas_call(
        _p1_body, grid=(nblk,),
        in_specs=[gspec, gspec],
        out_specs=pl.BlockSpec((B, 128), lambda i: (0, 0)),
        out_shape=jax.ShapeDtypeStruct((B, 128), jnp.float32),
    )(g1, b3)
    gd = pl.pallas_call(
        _p2_body, grid=(nblk,),
        in_specs=[gspec, gspec, fullspec((B, 128))],
        out_specs=pl.BlockSpec((B, 128), lambda i: (0, 0)),
        out_shape=jax.ShapeDtypeStruct((B, 128), jnp.float32),
    )(g1, b3, gmax)
    pooled = pl.pallas_call(
        _p3_body, grid=(nblk,),
        in_specs=[gspec, gspec, pl.BlockSpec((1000, HID), lambda i: (i, 0)),
                  fullspec((B, 128)), fullspec((B, 128))],
        out_specs=pl.BlockSpec((B, HID), lambda i: (0, 0)),
        out_shape=jax.ShapeDtypeStruct((B, HID), jnp.float32),
    )(g1, b3, h2r, gmax, gd)
    return pooled


def _sc_body(row0, row1, build, *refs):
    if build:
        (esrc, edst, aleT, asdT, xp, pb,
         hout, l_src, l_dl, l_eid, cnts,
         ls_src, ls_dl, ls_e0, ls_e1, den0, den1, pbv, cnt_ref,
         sem0, sem1, sem2, sem3) = refs
    else:
        (aleT, asdT, xp, pb, l_src, l_dl, l_eid, cnts,
         hout,
         ls_src, ls_dl, ls_e0, ls_e1, den0, den1, pbv, cnt_ref,
         sem0, sem1, sem2, sem3) = refs
    wid = lax.axis_index("s") * NC + lax.axis_index("c")
    lo = wid * R
    iota = lax.iota(jnp.int32, LN)
    zi = jnp.zeros((LN,), jnp.int32)
    zf = jnp.zeros((LN,), jnp.float32)

    pltpu.sync_copy(pb, pbv)

    def load_tables(als0, als1, ald0, ald1, dp0, dp1):
        pltpu.sync_copy(asdT.at[pl.ds(0 * NP, NP)], als0)
        pltpu.sync_copy(asdT.at[pl.ds(8 * NP, NP)], als1)
        pltpu.sync_copy(asdT.at[pl.ds(16 * NP, NP)], ald0)
        pltpu.sync_copy(asdT.at[pl.ds(24 * NP, NP)], ald1)

        def zdp(i, c):
            dp0[pl.ds(i * LN, LN)] = zf
            dp1[pl.ds(i * LN, LN)] = zf
            return c
        lax.fori_loop(0, LN * R // LN, zdp, 0)

    def den_reduce(dp0, dp1):
        def dred(c, z):
            o = c * LN
            t0 = dp0[pl.ds(o, LN)]
            t1 = dp1[pl.ds(o, LN)]
            for l in range(1, LN):
                t0 = t0 + dp0[pl.ds(l * R + o, LN)]
                t1 = t1 + dp1[pl.ds(l * R + o, LN)]
            den0[pl.ds(o, LN)] = t0
            den1[pl.ds(o, LN)] = t1
            return z
        lax.fori_loop(0, R // LN, dred, 0)

    def phase_a(als0, als1, ald0, ald1, dp0, dp1, ls_eid, v16,
                st_s0, st_s1, st_d0, st_d1, st_a00, st_a01, st_a10, st_a11):
        st_s = (st_s0, st_s1)
        st_d = (st_d0, st_d1)
        st_a0 = (st_a00, st_a01)
        st_a1 = (st_a10, st_a11)
        load_tables(als0, als1, ald0, ald1, dp0, dp1)

        def issue(ch, b):
            sem = sem0 if b == 0 else sem1
            pltpu.async_copy(esrc.at[pl.ds(ch * CH, CH)], st_s[b], sem)
            pltpu.async_copy(edst.at[pl.ds(ch * CH, CH)], st_d[b], sem)
            pltpu.async_copy(aleT.at[pl.ds(row0 * EP + ch * CH, CH)],
                             st_a0[b], sem)
            pltpu.async_copy(aleT.at[pl.ds(row1 * EP + ch * CH, CH)],
                             st_a1[b], sem)

        def wait(b):
            sem = sem0 if b == 0 else sem1
            pltpu.make_async_copy(esrc.at[pl.ds(0, CH)], st_s[b], sem).wait()
            pltpu.make_async_copy(edst.at[pl.ds(0, CH)], st_d[b], sem).wait()
            pltpu.make_async_copy(aleT.at[pl.ds(0, CH)], st_a0[b], sem).wait()
            pltpu.make_async_copy(aleT.at[pl.ds(0, CH)], st_a1[b], sem).wait()

        issue(0, 0)
        issue(1, 1)

        def pair_body(p, cnt):
            for b in (0, 1):
                ch = p * 2 + b
                wait(b)

                def vec_body(v, cnt):
                    srcv = st_s[b][pl.ds(v * LN, LN)]
                    dstv = st_d[b][pl.ds(v * LN, LN)]
                    a0v = st_a0[b][pl.ds(v * LN, LN)]
                    a1v = st_a1[b][pl.ds(v * LN, LN)]
                    eidv = ch * CH + v * LN + iota
                    mask = (dstv >= lo) & (dstv < lo + R)
                    dlv = dstv - lo
                    s0 = plsc.load_gather(als0, [srcv], mask=mask)
                    s1 = plsc.load_gather(als1, [srcv], mask=mask)
                    d0 = plsc.load_gather(ald0, [dstv], mask=mask)
                    d1 = plsc.load_gather(ald1, [dstv], mask=mask)
                    al0 = s0 + d0 + a0v
                    al1 = s1 + d1 + a1v
                    al0 = jnp.maximum(al0, 0.2 * al0)
                    al1 = jnp.maximum(al1, 0.2 * al1)
                    e0 = jnp.exp(al0)
                    e1 = jnp.exp(al1)
                    plsc.addupdate_scatter(dp0, [iota * R + dlv], e0, mask=mask)
                    plsc.addupdate_scatter(dp1, [iota * R + dlv], e1, mask=mask)
                    plsc.store_compressed(ls_src.at[pl.ds(cnt, LN)], srcv, mask=mask)
                    plsc.store_compressed(ls_dl.at[pl.ds(cnt, LN)], dlv, mask=mask)
                    plsc.store_compressed(ls_e0.at[pl.ds(cnt, LN)], e0, mask=mask)
                    plsc.store_compressed(ls_e1.at[pl.ds(cnt, LN)], e1, mask=mask)
                    plsc.store_compressed(ls_eid.at[pl.ds(cnt, LN)], eidv,
                                          mask=mask)
                    return cnt + jnp.sum(mask.astype(jnp.int32))

                cnt = lax.fori_loop(0, CH // LN, vec_body, cnt)

                @pl.when(ch + 2 < NCH)
                def _():
                    issue(ch + 2, b)
            return cnt

        cnt = lax.fori_loop(0, NCH // 2, pair_body, 0)

        for padv in (0, LN):
            ls_src[pl.ds(cnt + padv, LN)] = zi
            ls_dl[pl.ds(cnt + padv, LN)] = zi
            ls_e0[pl.ds(cnt + padv, LN)] = zf
            ls_e1[pl.ds(cnt + padv, LN)] = zf
            ls_eid[pl.ds(cnt + padv, LN)] = zi
        cnt_ref[0] = cnt

        pltpu.sync_copy(ls_src, l_src.at[pl.ds(wid * CAP, CAP)])
        pltpu.sync_copy(ls_dl, l_dl.at[pl.ds(wid * CAP, CAP)])
        pltpu.sync_copy(ls_eid, l_eid.at[pl.ds(wid * CAP, CAP)])
        v16[pl.ds(0, LN)] = jnp.where(iota == 0, cnt, 0)
        pltpu.sync_copy(v16, cnts.at[pl.ds(wid * LN, LN)])

        den_reduce(dp0, dp1)

    def phase_a2(als0, als1, ald0, ald1, dp0, dp1, ale0o, ale1o, tmpi, v16):
        pltpu.sync_copy(l_src.at[pl.ds(wid * CAP, CAP)], ls_src)
        pltpu.sync_copy(l_dl.at[pl.ds(wid * CAP, CAP)], ls_dl)
        pltpu.sync_copy(cnts.at[pl.ds(wid * LN, LN)], v16)
        cnt = v16[pl.ds(0, LN)][0]
        cnt_ref[0] = cnt
        load_tables(als0, als1, ald0, ald1, dp0, dp1)

        pltpu.sync_copy(l_eid.at[pl.ds(wid * CAP, CAP)], tmpi)

        def bld(g, z):
            o = g * LN
            ev = tmpi[pl.ds(o, LN)]
            valid = (o + iota) < cnt
            tmpi[pl.ds(o, LN)] = jnp.where(valid, ev + row0 * EP, o + iota)
            return z
        lax.fori_loop(0, CAP // LN, bld, 0)
        pltpu.async_copy(aleT.at[tmpi], ale0o, sem0).wait()

        def bld2(g, z):
            o = g * LN
            tmpi[pl.ds(o, LN)] = tmpi[pl.ds(o, LN)] + (row1 - row0) * EP
            return z
        lax.fori_loop(0, CAP // LN, bld2, 0)
        pltpu.async_copy(aleT.at[tmpi], ale1o, sem0).wait()

        cnt16 = (cnt + LN - 1) // LN

        def lb(g, z):
            o = g * LN
            srcv = ls_src[pl.ds(o, LN)]
            dlv = ls_dl[pl.ds(o, LN)]
            a0v = ale0o[pl.ds(o, LN)]
            a1v = ale1o[pl.ds(o, LN)]
            valid = (o + iota) < cnt
            dstv = dlv + lo
            s0 = plsc.load_gather(als0, [srcv], mask=valid)
            s1 = plsc.load_gather(als1, [srcv], mask=valid)
            d0 = plsc.load_gather(ald0, [dstv], mask=valid)
            d1 = plsc.load_gather(ald1, [dstv], mask=valid)
            al0 = s0 + d0 + a0v
            al1 = s1 + d1 + a1v
            al0 = jnp.maximum(al0, 0.2 * al0)
            al1 = jnp.maximum(al1, 0.2 * al1)
            e0 = jnp.where(valid, jnp.exp(al0), 0.0)
            e1 = jnp.where(valid, jnp.exp(al1), 0.0)
            plsc.addupdate_scatter(dp0, [iota * R + dlv], e0, mask=valid)
            plsc.addupdate_scatter(dp1, [iota * R + dlv], e1, mask=valid)
            ls_e0[pl.ds(o, LN)] = e0
            ls_e1[pl.ds(o, LN)] = e1
            return z
        lax.fori_loop(0, cnt16, lb, 0)
        ls_e0[pl.ds(cnt16 * LN, LN)] = zf
        ls_e1[pl.ds(cnt16 * LN, LN)] = zf
        den_reduce(dp0, dp1)

    if build:
        pl.run_scoped(
            phase_a,
            pltpu.VMEM((NP,), jnp.float32),
            pltpu.VMEM((NP,), jnp.float32),
            pltpu.VMEM((NP,), jnp.float32),
            pltpu.VMEM((NP,), jnp.float32),
            pltpu.VMEM((LN * R,), jnp.float32),
            pltpu.VMEM((LN * R,), jnp.float32),
            pltpu.VMEM((CAP,), jnp.int32),
            pltpu.VMEM((LN,), jnp.int32),
            pltpu.VMEM((CH,), jnp.int32),
            pltpu.VMEM((CH,), jnp.int32),
            pltpu.VMEM((CH,), jnp.int32),
            pltpu.VMEM((CH,), jnp.int32),
            pltpu.VMEM((CH,), jnp.float32),
            pltpu.VMEM((CH,), jnp.float32),
            pltpu.VMEM((CH,), jnp.float32),
            pltpu.VMEM((CH,), jnp.float32),
        )
    else:
        pl.run_scoped(
            phase_a2,
            pltpu.VMEM((NP,), jnp.float32),
            pltpu.VMEM((NP,), jnp.float32),
            pltpu.VMEM((NP,), jnp.float32),
            pltpu.VMEM((NP,), jnp.float32),
            pltpu.VMEM((LN * R,), jnp.float32),
            pltpu.VMEM((LN * R,), jnp.float32),
            pltpu.VMEM((CAP,), jnp.float32),
            pltpu.VMEM((CAP,), jnp.float32),
            pltpu.VMEM((CAP,), jnp.int32),
            pltpu.VMEM((LN,), jnp.int32),
        )

    cnt = cnt_ref[0]
    nvec = (cnt + LN - 1) // LN

    def norm_body(g, z):
        o = g * LN
        dlv = ls_dl[pl.ds(o, LN)]
        e0 = ls_e0[pl.ds(o, LN)]
        e1 = ls_e1[pl.ds(o, LN)]
        d0 = plsc.load_gather(den0, [dlv])
        d1 = plsc.load_gather(den1, [dlv])
        ls_e0[pl.ds(o, LN)] = e0 * 0.5 / (d0 + 1e-16)
        ls_e1[pl.ds(o, LN)] = e1 * 0.5 / (d1 + 1e-16)
        return z
    lax.fori_loop(0, nvec, norm_body, 0)

    def phase_b(acc, rows0, rows1):
        rows = (rows0, rows1)
        sems = (sem0, sem1)

        def zacc(r, z):
            for c in range(HID // LN):
                acc[r, pl.ds(c * LN, LN)] = zf
            return z
        lax.fori_loop(0, R, zacc, 0)

        nbp = (nvec + 1) // 2

        def issue(g, b):
            pltpu.async_copy(xp.at[ls_src.at[pl.ds(g * GK, GK)]], rows[b],
                             sems[b])

        def wait(b):
            pltpu.make_async_copy(xp.at[ls_src.at[pl.ds(0, GK)]], rows[b],
                                  sems[b]).wait()

        for b0 in range(2):
            @pl.when(nbp > b0)
            def _(b0=b0):
                issue(b0, b0)

        def pairs(p, z):
            for b in range(2):
                g = p * 2 + b

                @pl.when(g < nbp)
                def _():
                    wait(b)
                    for h in range(2):
                        dlv = ls_dl[pl.ds(g * GK + h * LN, LN)]
                        a0v = ls_e0[pl.ds(g * GK + h * LN, LN)]
                        a1v = ls_e1[pl.ds(g * GK + h * LN, LN)]
                        for j in range(LN):
                            dl = dlv[j]
                            a0 = a0v[j]
                            a1 = a1v[j]
                            jr = h * LN + j
                            for c2 in range(HID // 32):
                                u0, u1 = plsc.unpack(
                                    plsc.bitcast(
                                        rows[b][jr, pl.ds(c2 * LN, LN)],
                                        jnp.bfloat16),
                                    format=plsc.PackFormat.INTERLEAVED,
                                    preferred_element_type=jnp.float32)
                                w0, w1 = plsc.unpack(
                                    plsc.bitcast(
                                        rows[b][jr, pl.ds(HID // 2 + c2 * LN,
                                                          LN)],
                                        jnp.bfloat16),
                                    format=plsc.PackFormat.INTERLEAVED,
                                    preferred_element_type=jnp.float32)
                                plsc.addupdate(
                                    acc.at[dl, pl.ds(c2 * 32, LN)],
                                    u0 * a0 + w0 * a1)
                                plsc.addupdate(
                                    acc.at[dl, pl.ds(c2 * 32 + LN, LN)],
                                    u1 * a0 + w1 * a1)

                    @pl.when(g + 2 < nbp)
                    def _():
                        issue(g + 2, b)
            return z

        lax.fori_loop(0, (nbp + 1) // 2, pairs, 0)

        pcoef = pbv[pl.ds(0, LN)][0]

        def fin(r, z):
            for c in range(HID // LN):
                bc = pbv[pl.ds(LN + c * LN, LN)]
                v = acc[r, pl.ds(c * LN, LN)] + bc
                v = jnp.where(v >= 0.0, v, pcoef * v)
                acc[r, pl.ds(c * LN, LN)] = v
            return z
        lax.fori_loop(0, R, fin, 0)
        pltpu.sync_copy(acc, hout.at[pl.ds(lo, R), :])

    pl.run_scoped(
        phase_b,
        pltpu.VMEM((R, HID), jnp.float32),
        pltpu.VMEM((GK, HID), jnp.int32),
        pltpu.VMEM((GK, HID), jnp.int32),
    )


def _make_sc_layer(row0, row1, build):
    mesh = plsc.VectorSubcoreMesh(core_axis_name="c", subcore_axis_name="s",
                                  num_cores=NC, num_subcores=NS)
    if build:
        out_type = [
            jax.ShapeDtypeStruct((NP, HID), jnp.float32),
            jax.ShapeDtypeStruct((NW * CAP,), jnp.int32),
            jax.ShapeDtypeStruct((NW * CAP,), jnp.int32),
            jax.ShapeDtypeStruct((NW * CAP,), jnp.int32),
            jax.ShapeDtypeStruct((NW * LN,), jnp.int32),
        ]
    else:
        out_type = jax.ShapeDtypeStruct((NP, HID), jnp.float32)
    return pl.kernel(
        functools.partial(_sc_body, row0, row1, build),
        out_type=out_type,
        mesh=mesh,
        compiler_params=pltpu.CompilerParams(needs_layout_passes=False),
        scratch_types=[
            pltpu.VMEM((CAP,), jnp.int32),
            pltpu.VMEM((CAP,), jnp.int32),
            pltpu.VMEM((CAP,), jnp.float32),
            pltpu.VMEM((CAP,), jnp.float32),
            pltpu.VMEM((R,), jnp.float32),
            pltpu.VMEM((R,), jnp.float32),
            pltpu.VMEM((272,), jnp.float32),
            pltpu.SMEM((1,), jnp.int32),
            pltpu.SemaphoreType.DMA,
            pltpu.SemaphoreType.DMA,
            pltpu.SemaphoreType.DMA,
            pltpu.SemaphoreType.DMA,
        ],
    )


def _edge_phase_jnp(esrc, edst, aleT, asdT, xp, pb, row0, row1):
    src = esrc[:E]
    dst = edst[:E]
    als = asdT[(0, 8), :].T
    ald = asdT[(16, 24), :].T
    ale = aleT[(row0, row1), :E].T
    al = als[src] + ald[dst] + ale
    al = jnp.maximum(al, 0.2 * al)
    e = jnp.exp(al)
    den = jax.ops.segment_sum(e, dst, num_segments=NP)
    alpha = e / (den[dst] + 1e-16) * 0.5
    acc = jnp.zeros((NP, HID))
    for h in range(H):
        acc = acc + jax.ops.segment_sum(
            xp[src, h * HID:(h + 1) * HID] * alpha[:, h:h + 1],
            dst, num_segments=NP)
    out = acc + pb[LN:LN + HID]
    return jnp.where(out >= 0, out, pb[0] * out)


_USE_SC = True


def _fold(W, a):
    C = a.shape[1]
    cols = [W[:, h * C:(h + 1) * C] @ a[h] for h in range(a.shape[0])]
    return jnp.stack(cols, axis=0)


def kernel(x, edge_index, edge_attr, batch, W1, We1, as1, ad1, ae1, b1, p1,
           W2, We2, as2, ad2, ae2, b2, p2, G1, gb1, gp1, G2, gb2, gp2, G3, gb3):
    xpad = jnp.pad(x, ((0, NP - NREAL), (0, 0)))
    esrc = jnp.pad(edge_index[0], (0, EP - E))
    edst = jnp.pad(edge_index[1], (0, EP - E), constant_values=-1)
    ea_pad = jnp.pad(edge_attr, ((0, EP - E), (0, 0)))

    def _spread(rows, K):
        out = jnp.zeros((32, K))
        for i, r in enumerate(rows):
            out = out.at[8 * i].set(r)
        return out

    f1s, f1d = _fold(W1, as1), _fold(W1, ad1)
    f2s, f2d = _fold(W2, as2), _fold(W2, ad2)
    fe1, fe2 = _fold(We1, ae1), _fold(We2, ae2)
    wsd1T = _spread([f1s[0], f1s[1], f1d[0], f1d[1]], CLIP)
    wsd2T = _spread([f2s[0], f2s[1], f2d[0], f2d[1]], HID)
    weT = _spread([fe1[0], fe1[1], fe2[0], fe2[1]], CLIP)
    pb1 = jnp.concatenate([p1[None], jnp.zeros(15), b1])
    pb2 = jnp.concatenate([p2[None], jnp.zeros(15), b2])

    perm = jnp.arange(H * HID).reshape(16, 2, 16).transpose(0, 2, 1).reshape(-1)
    W1p = W1[:, perm]
    W2p = W2[:, perm]

    aleT = _edge_logits(ea_pad, weT)
    xp1, asd1T = _proj(xpad, W1p, wsd1T)
    aleT_f = aleT.reshape(-1)
    xp1u = lax.bitcast_convert_type(xp1.reshape(NP, HID, 2), jnp.int32)
    if _USE_SC:
        h1, l_src, l_dl, l_eid, cnts = _make_sc_layer(0, 8, True)(
            esrc, edst, aleT_f, asd1T.reshape(-1), xp1u, pb1)
    else:
        h1 = _edge_phase_jnp(esrc, edst, aleT, asd1T, xp1, pb1, 0, 8)
    xp2, asd2T = _proj(h1, W2p, wsd2T)
    xp2u = lax.bitcast_convert_type(xp2.reshape(NP, HID, 2), jnp.int32)
    if _USE_SC:
        h2 = _make_sc_layer(16, 24, False)(
            aleT_f, asd2T.reshape(-1), xp2u, pb2, l_src, l_dl, l_eid, cnts)
    else:
        h2 = _edge_phase_jnp(esrc, edst, aleT, asd2T, xp2, pb2, 16, 24)

    g = _mlp(h2, G1, gb1, gp1, G2, gb2, gp2, G3, gb3)
    g1 = g[:NREAL, 0].reshape(10, 1, 1000)
    b3 = batch.reshape(10, 1, 1000)
    h2r = h2[:NREAL]
    pooled = _pool(g1, b3, h2r)
    return pooled.reshape(B, PRE, MB)


# speedup vs baseline: 1.0087x; 1.0087x over previous
import functools

import jax
import jax.numpy as jnp
from jax import lax
from jax.experimental import pallas as pl
from jax.experimental.pallas import tpu as pltpu
from jax.experimental.pallas import tpu_sc as plsc

NREAL = 10000
E = 160000
CLIP = 128
H = 2
HID = 256
PRE = 2
MB = 128
OUT2 = PRE * MB
B = 16

NC = 2
NS = 16
NW = NC * NS
LN = 16
NP = 10240
R = NP // NW
EP = 163840
CH = 2048
NCH = EP // CH
CAP = 6160
GK = 16


def _proj_body(x_ref, w_ref, wsdt_ref, xp_ref, asdt_ref):
    xb = x_ref[...]
    xp_ref[...] = jnp.dot(
        xb, w_ref[...], preferred_element_type=jnp.float32
    ).astype(jnp.bfloat16)
    asdt_ref[...] = lax.dot_general(
        wsdt_ref[...], xb, (((1,), (1,)), ((), ())),
        preferred_element_type=jnp.float32)


def _proj(xpad, W, wsdT):
    K = xpad.shape[1]
    HW = W.shape[1]
    blk = 1024
    return pl.pallas_call(
        _proj_body,
        grid=(NP // blk,),
        in_specs=[
            pl.BlockSpec((blk, K), lambda i: (i, 0)),
            pl.BlockSpec((K, HW), lambda i: (0, 0)),
            pl.BlockSpec((32, K), lambda i: (0, 0)),
        ],
        out_specs=[
            pl.BlockSpec((blk, HW), lambda i: (i, 0)),
            pl.BlockSpec((32, blk), lambda i: (0, i)),
        ],
        out_shape=[
            jax.ShapeDtypeStruct((NP, HW), jnp.bfloat16),
            jax.ShapeDtypeStruct((32, NP), jnp.float32),
        ],
    )(xpad, W, wsdT)


def _edge_body(ea_ref, wet_ref, out_ref):
    out_ref[...] = lax.dot_general(
        wet_ref[...], ea_ref[...], (((1,), (1,)), ((), ())),
        preferred_element_type=jnp.float32)


def _edge_logits(ea_pad, weT):
    blk = 2048
    return pl.pallas_call(
        _edge_body,
        grid=(EP // blk,),
        in_specs=[
            pl.BlockSpec((blk, CLIP), lambda i: (i, 0)),
            pl.BlockSpec((32, CLIP), lambda i: (0, 0)),
        ],
        out_specs=pl.BlockSpec((32, blk), lambda i: (0, i)),
        out_shape=jax.ShapeDtypeStruct((32, EP), jnp.float32),
    )(ea_pad, weT)


def _mlp_body(h_ref, g1_ref, gb1_ref, g2_ref, gb2_ref, g3_ref, ps_ref, out_ref):
    h = h_ref[...]
    gp1 = ps_ref[0, 0]
    gp2 = ps_ref[0, 1]
    gb3 = ps_ref[0, 2]
    t = jnp.dot(h, g1_ref[...], preferred_element_type=jnp.float32) + gb1_ref[...]
    t = jnp.where(t >= 0, t, gp1 * t)
    t = jnp.dot(t, g2_ref[...], preferred_element_type=jnp.float32) + gb2_ref[...]
    t = jnp.where(t >= 0, t, gp2 * t)
    out_ref[...] = jnp.dot(t, g3_ref[...], preferred_element_type=jnp.float32) + gb3


def _mlp(h, G1, gb1, gp1, G2, gb2, gp2, G3, gb3):
    g3p = jnp.pad(G3, ((0, 0), (0, 127)))
    ps = jnp.stack([gp1, gp2, gb3[0]]).reshape(1, 3)
    blk = 1024
    out = pl.pallas_call(
        _mlp_body,
        grid=(NP // blk,),
        in_specs=[
            pl.BlockSpec((blk, HID), lambda i: (i, 0)),
            pl.BlockSpec((HID, HID), lambda i: (0, 0)),
            pl.BlockSpec((1, HID), lambda i: (0, 0)),
            pl.BlockSpec((HID, HID), lambda i: (0, 0)),
            pl.BlockSpec((1, HID), lambda i: (0, 0)),
            pl.BlockSpec((HID, 128), lambda i: (0, 0)),
            pl.BlockSpec((1, 3), lambda i: (0, 0)),
        ],
        out_specs=pl.BlockSpec((blk, 128), lambda i: (i, 0)),
        out_shape=jax.ShapeDtypeStruct((NP, 128), jnp.float32),
    )(h, G1, gb1.reshape(1, HID), G2, gb2.reshape(1, HID), g3p, ps)
    return out


def _p1_body(g_ref, b_ref, o_ref):
    i = pl.program_id(0)

    @pl.when(i == 0)
    def _():
        o_ref[...] = jnp.full((B, 128), -1e30, jnp.float32)

    g_row = g_ref[0]
    bat = b_ref[0]
    M = bat == lax.broadcasted_iota(jnp.int32, (B, 1), 0)
    masked = jnp.where(M, g_row, -1e30)
    cur = jnp.max(masked, axis=1, keepdims=True)
    o_ref[...] = jnp.maximum(o_ref[...], jnp.broadcast_to(cur, (B, 128)))


def _p2_body(g_ref, b_ref, gm_ref, o_ref):
    i = pl.program_id(0)

    @pl.when(i == 0)
    def _():
        o_ref[...] = jnp.zeros((B, 128), jnp.float32)

    g_row = g_ref[0]
    bat = b_ref[0]
    M = bat == lax.broadcasted_iota(jnp.int32, (B, 1), 0)
    gmn = jnp.sum(jnp.where(M, gm_ref[:, 0:1], 0.0), axis=0, keepdims=True)
    ge = jnp.exp(g_row - gmn)
    cur = jnp.sum(M.astype(jnp.float32) * ge, axis=1, keepdims=True)
    o_ref[...] = o_ref[...] + jnp.broadcast_to(cur, (B, 128))


def _p3_body(g_ref, b_ref, h_ref, gm_ref, gd_ref, o_ref):
    i = pl.program_id(0)

    @pl.when(i == 0)
    def _():
        o_ref[...] = jnp.zeros((B, HID), jnp.float32)

    g_row = g_ref[0]
    bat = b_ref[0]
    M = bat == lax.broadcasted_iota(jnp.int32, (B, 1), 0)
    gmn = jnp.sum(jnp.where(M, gm_ref[:, 0:1], 0.0), axis=0, keepdims=True)
    gdn = jnp.sum(jnp.where(M, gd_ref[:, 0:1], 0.0), axis=0, keepdims=True)
    ge = jnp.exp(g_row - gmn)
    alpha = ge / (gdn + 1e-16)
    S = M.astype(jnp.float32) * alpha
    o_ref[...] = o_ref[...] + jnp.dot(S, h_ref[...],
                                      preferred_element_type=jnp.float32)


def _pool(g1, b3, h2r):
    nblk = 10
    gspec = pl.BlockSpec((1, 1, 1000), lambda i: (i, 0, 0))
    fullspec = lambda shp: pl.BlockSpec(shp, lambda i: (0, 0))
    gmax = pl.pallas_call(
        _p1_body, grid=(nblk,),
        in_specs=[gspec, gspec],
        out_specs=pl.BlockSpec((B, 128), lambda i: (0, 0)),
        out_shape=jax.ShapeDtypeStruct((B, 128), jnp.float32),
    )(g1, b3)
    gd = pl.pallas_call(
        _p2_body, grid=(nblk,),
        in_specs=[gspec, gspec, fullspec((B, 128))],
        out_specs=pl.BlockSpec((B, 128), lambda i: (0, 0)),
        out_shape=jax.ShapeDtypeStruct((B, 128), jnp.float32),
    )(g1, b3, gmax)
    pooled = pl.pallas_call(
        _p3_body, grid=(nblk,),
        in_specs=[gspec, gspec, pl.BlockSpec((1000, HID), lambda i: (i, 0)),
                  fullspec((B, 128)), fullspec((B, 128))],
        out_specs=pl.BlockSpec((B, HID), lambda i: (0, 0)),
        out_shape=jax.ShapeDtypeStruct((B, HID), jnp.float32),
    )(g1, b3, h2r, gmax, gd)
    return pooled


def _sc_body(row0, row1, build, *refs):
    if build:
        (esrc, edst, aleT, asdT, xp, pb,
         hout, l_src, l_dl, l_eid, cnts,
         ls_src, ls_dl, ls_e0, ls_e1, den0, den1, pbv, cnt_ref,
         sem0, sem1, sem2, sem3) = refs
    else:
        (aleT, asdT, xp, pb, l_src, l_dl, l_eid, cnts,
         hout,
         ls_src, ls_dl, ls_e0, ls_e1, den0, den1, pbv, cnt_ref,
         sem0, sem1, sem2, sem3) = refs
    wid = lax.axis_index("s") * NC + lax.axis_index("c")
    lo = wid * R
    iota = lax.iota(jnp.int32, LN)
    zi = jnp.zeros((LN,), jnp.int32)
    zf = jnp.zeros((LN,), jnp.float32)

    pltpu.sync_copy(pb, pbv)

    def load_tables(als0, als1, ald0, ald1, dp0, dp1):
        pltpu.sync_copy(asdT.at[pl.ds(0 * NP, NP)], als0)
        pltpu.sync_copy(asdT.at[pl.ds(8 * NP, NP)], als1)
        pltpu.sync_copy(asdT.at[pl.ds(16 * NP, NP)], ald0)
        pltpu.sync_copy(asdT.at[pl.ds(24 * NP, NP)], ald1)

        def zdp(i, c):
            dp0[pl.ds(i * LN, LN)] = zf
            dp1[pl.ds(i * LN, LN)] = zf
            return c
        lax.fori_loop(0, LN * R // LN, zdp, 0)

    def den_reduce(dp0, dp1):
        def dred(c, z):
            o = c * LN
            t0 = dp0[pl.ds(o, LN)]
            t1 = dp1[pl.ds(o, LN)]
            for l in range(1, LN):
                t0 = t0 + dp0[pl.ds(l * R + o, LN)]
                t1 = t1 + dp1[pl.ds(l * R + o, LN)]
            den0[pl.ds(o, LN)] = t0
            den1[pl.ds(o, LN)] = t1
            return z
        lax.fori_loop(0, R // LN, dred, 0)

    def phase_a(als0, als1, ald0, ald1, dp0, dp1, ls_eid, v16,
                st_s0, st_s1, st_d0, st_d1, st_a00, st_a01, st_a10, st_a11):
        st_s = (st_s0, st_s1)
        st_d = (st_d0, st_d1)
        st_a0 = (st_a00, st_a01)
        st_a1 = (st_a10, st_a11)
        load_tables(als0, als1, ald0, ald1, dp0, dp1)

        def issue(ch, b):
            sem = sem0 if b == 0 else sem1
            pltpu.async_copy(esrc.at[pl.ds(ch * CH, CH)], st_s[b], sem)
            pltpu.async_copy(edst.at[pl.ds(ch * CH, CH)], st_d[b], sem)
            pltpu.async_copy(aleT.at[pl.ds(row0 * EP + ch * CH, CH)],
                             st_a0[b], sem)
            pltpu.async_copy(aleT.at[pl.ds(row1 * EP + ch * CH, CH)],
                             st_a1[b], sem)

        def wait(b):
            sem = sem0 if b == 0 else sem1
            pltpu.make_async_copy(esrc.at[pl.ds(0, CH)], st_s[b], sem).wait()
            pltpu.make_async_copy(edst.at[pl.ds(0, CH)], st_d[b], sem).wait()
            pltpu.make_async_copy(aleT.at[pl.ds(0, CH)], st_a0[b], sem).wait()
            pltpu.make_async_copy(aleT.at[pl.ds(0, CH)], st_a1[b], sem).wait()

        issue(0, 0)
        issue(1, 1)

        def pair_body(p, cnt):
            for b in (0, 1):
                ch = p * 2 + b
                wait(b)

                def vec_body(v, cnt):
                    srcv = st_s[b][pl.ds(v * LN, LN)]
                    dstv = st_d[b][pl.ds(v * LN, LN)]
                    a0v = st_a0[b][pl.ds(v * LN, LN)]
                    a1v = st_a1[b][pl.ds(v * LN, LN)]
                    eidv = ch * CH + v * LN + iota
                    mask = (dstv >= lo) & (dstv < lo + R)
                    dlv = dstv - lo
                    s0 = plsc.load_gather(als0, [srcv], mask=mask)
                    s1 = plsc.load_gather(als1, [srcv], mask=mask)
                    d0 = plsc.load_gather(ald0, [dstv], mask=mask)
                    d1 = plsc.load_gather(ald1, [dstv], mask=mask)
                    al0 = s0 + d0 + a0v
                    al1 = s1 + d1 + a1v
                    al0 = jnp.maximum(al0, 0.2 * al0)
                    al1 = jnp.maximum(al1, 0.2 * al1)
                    e0 = jnp.exp(al0)
                    e1 = jnp.exp(al1)
                    plsc.addupdate_scatter(dp0, [iota * R + dlv], e0, mask=mask)
                    plsc.addupdate_scatter(dp1, [iota * R + dlv], e1, mask=mask)
                    plsc.store_compressed(ls_src.at[pl.ds(cnt, LN)], srcv, mask=mask)
                    plsc.store_compressed(ls_dl.at[pl.ds(cnt, LN)], dlv, mask=mask)
                    plsc.store_compressed(ls_e0.at[pl.ds(cnt, LN)], e0, mask=mask)
                    plsc.store_compressed(ls_e1.at[pl.ds(cnt, LN)], e1, mask=mask)
                    plsc.store_compressed(ls_eid.at[pl.ds(cnt, LN)], eidv,
                                          mask=mask)
                    return cnt + jnp.sum(mask.astype(jnp.int32))

                cnt = lax.fori_loop(0, CH // LN, vec_body, cnt)

                @pl.when(ch + 2 < NCH)
                def _():
                    issue(ch + 2, b)
            return cnt

        cnt = lax.fori_loop(0, NCH // 2, pair_body, 0)

        for padv in (0, LN):
            ls_src[pl.ds(cnt + padv, LN)] = zi
            ls_dl[pl.ds(cnt + padv, LN)] = zi
            ls_e0[pl.ds(cnt + padv, LN)] = zf
            ls_e1[pl.ds(cnt + padv, LN)] = zf
            ls_eid[pl.ds(cnt + padv, LN)] = zi
        cnt_ref[0] = cnt

        pltpu.sync_copy(ls_src, l_src.at[pl.ds(wid * CAP, CAP)])
        pltpu.sync_copy(ls_dl, l_dl.at[pl.ds(wid * CAP, CAP)])
        pltpu.sync_copy(ls_eid, l_eid.at[pl.ds(wid * CAP, CAP)])
        v16[pl.ds(0, LN)] = jnp.where(iota == 0, cnt, 0)
        pltpu.sync_copy(v16, cnts.at[pl.ds(wid * LN, LN)])

        den_reduce(dp0, dp1)

    def phase_a2(als0, als1, ald0, ald1, dp0, dp1, ale0o, ale1o, tmpi, v16):
        pltpu.sync_copy(l_src.at[pl.ds(wid * CAP, CAP)], ls_src)
        pltpu.sync_copy(l_dl.at[pl.ds(wid * CAP, CAP)], ls_dl)
        pltpu.sync_copy(cnts.at[pl.ds(wid * LN, LN)], v16)
        cnt = v16[pl.ds(0, LN)][0]
        cnt_ref[0] = cnt
        load_tables(als0, als1, ald0, ald1, dp0, dp1)

        pltpu.sync_copy(l_eid.at[pl.ds(wid * CAP, CAP)], tmpi)

        def bld(g, z):
            o = g * LN
            ev = tmpi[pl.ds(o, LN)]
            valid = (o + iota) < cnt
            tmpi[pl.ds(o, LN)] = jnp.where(valid, ev + row0 * EP, o + iota)
            return z
        lax.fori_loop(0, CAP // LN, bld, 0)
        pltpu.async_copy(aleT.at[tmpi], ale0o, sem0).wait()

        def bld2(g, z):
            o = g * LN
            tmpi[pl.ds(o, LN)] = tmpi[pl.ds(o, LN)] + (row1 - row0) * EP
            return z
        lax.fori_loop(0, CAP // LN, bld2, 0)
        pltpu.async_copy(aleT.at[tmpi], ale1o, sem0).wait()

        cnt16 = (cnt + LN - 1) // LN

        def lb(g, z):
            o = g * LN
            srcv = ls_src[pl.ds(o, LN)]
            dlv = ls_dl[pl.ds(o, LN)]
            a0v = ale0o[pl.ds(o, LN)]
            a1v = ale1o[pl.ds(o, LN)]
            valid = (o + iota) < cnt
            dstv = dlv + lo
            s0 = plsc.load_gather(als0, [srcv], mask=valid)
            s1 = plsc.load_gather(als1, [srcv], mask=valid)
            d0 = plsc.load_gather(ald0, [dstv], mask=valid)
            d1 = plsc.load_gather(ald1, [dstv], mask=valid)
            al0 = s0 + d0 + a0v
            al1 = s1 + d1 + a1v
            al0 = jnp.maximum(al0, 0.2 * al0)
            al1 = jnp.maximum(al1, 0.2 * al1)
            e0 = jnp.where(valid, jnp.exp(al0), 0.0)
            e1 = jnp.where(valid, jnp.exp(al1), 0.0)
            plsc.addupdate_scatter(dp0, [iota * R + dlv], e0, mask=valid)
            plsc.addupdate_scatter(dp1, [iota * R + dlv], e1, mask=valid)
            ls_e0[pl.ds(o, LN)] = e0
            ls_e1[pl.ds(o, LN)] = e1
            return z
        lax.fori_loop(0, cnt16, lb, 0)
        ls_e0[pl.ds(cnt16 * LN, LN)] = zf
        ls_e1[pl.ds(cnt16 * LN, LN)] = zf
        den_reduce(dp0, dp1)

    if build:
        pl.run_scoped(
            phase_a,
            pltpu.VMEM((NP,), jnp.float32),
            pltpu.VMEM((NP,), jnp.float32),
            pltpu.VMEM((NP,), jnp.float32),
            pltpu.VMEM((NP,), jnp.float32),
            pltpu.VMEM((LN * R,), jnp.float32),
            pltpu.VMEM((LN * R,), jnp.float32),
            pltpu.VMEM((CAP,), jnp.int32),
            pltpu.VMEM((LN,), jnp.int32),
            pltpu.VMEM((CH,), jnp.int32),
            pltpu.VMEM((CH,), jnp.int32),
            pltpu.VMEM((CH,), jnp.int32),
            pltpu.VMEM((CH,), jnp.int32),
            pltpu.VMEM((CH,), jnp.float32),
            pltpu.VMEM((CH,), jnp.float32),
            pltpu.VMEM((CH,), jnp.float32),
            pltpu.VMEM((CH,), jnp.float32),
        )
    else:
        pl.run_scoped(
            phase_a2,
            pltpu.VMEM((NP,), jnp.float32),
            pltpu.VMEM((NP,), jnp.float32),
            pltpu.VMEM((NP,), jnp.float32),
            pltpu.VMEM((NP,), jnp.float32),
            pltpu.VMEM((LN * R,), jnp.float32),
            pltpu.VMEM((LN * R,), jnp.float32),
            pltpu.VMEM((CAP,), jnp.float32),
            pltpu.VMEM((CAP,), jnp.float32),
            pltpu.VMEM((CAP,), jnp.int32),
            pltpu.VMEM((LN,), jnp.int32),
        )

    cnt = cnt_ref[0]
    nvec = (cnt + LN - 1) // LN

    def norm_body(g, z):
        o = g * LN
        dlv = ls_dl[pl.ds(o, LN)]
        e0 = ls_e0[pl.ds(o, LN)]
        e1 = ls_e1[pl.ds(o, LN)]
        d0 = plsc.load_gather(den0, [dlv])
        d1 = plsc.load_gather(den1, [dlv])
        ls_e0[pl.ds(o, LN)] = e0 * 0.5 / (d0 + 1e-16)
        ls_e1[pl.ds(o, LN)] = e1 * 0.5 / (d1 + 1e-16)
        return z
    lax.fori_loop(0, nvec, norm_body, 0)

    def phase_b(acc, rows0, rows1, rows2, rows3):
        rows = (rows0, rows1, rows2, rows3)
        sems = (sem0, sem1, sem2, sem3)

        def zacc(r, z):
            for c in range(HID // LN):
                acc[r, pl.ds(c * LN, LN)] = zf
            return z
        lax.fori_loop(0, R, zacc, 0)

        nbp = nvec

        def issue(g, b):
            pltpu.async_copy(xp.at[ls_src.at[pl.ds(g * GK, GK)]], rows[b],
                             sems[b])

        def wait(b):
            pltpu.make_async_copy(xp.at[ls_src.at[pl.ds(0, GK)]], rows[b],
                                  sems[b]).wait()

        for b0 in range(4):
            @pl.when(nbp > b0)
            def _(b0=b0):
                issue(b0, b0)

        def quads(p, z):
            for b in range(4):
                g = p * 4 + b

                @pl.when(g < nbp)
                def _():
                    wait(b)
                    dlv = ls_dl[pl.ds(g * GK, LN)]
                    a0v = ls_e0[pl.ds(g * GK, LN)]
                    a1v = ls_e1[pl.ds(g * GK, LN)]
                    for j in range(GK):
                        dl = dlv[j]
                        a0 = a0v[j]
                        a1 = a1v[j]
                        for c2 in range(HID // 32):
                            u0, u1 = plsc.unpack(
                                plsc.bitcast(
                                    rows[b][j, pl.ds(c2 * LN, LN)],
                                    jnp.bfloat16),
                                format=plsc.PackFormat.INTERLEAVED,
                                preferred_element_type=jnp.float32)
                            w0, w1 = plsc.unpack(
                                plsc.bitcast(
                                    rows[b][j, pl.ds(HID // 2 + c2 * LN, LN)],
                                    jnp.bfloat16),
                                format=plsc.PackFormat.INTERLEAVED,
                                preferred_element_type=jnp.float32)
                            plsc.addupdate(acc.at[dl, pl.ds(c2 * 32, LN)],
                                           u0 * a0 + w0 * a1)
                            plsc.addupdate(acc.at[dl, pl.ds(c2 * 32 + LN, LN)],
                                           u1 * a0 + w1 * a1)

                    @pl.when(g + 4 < nbp)
                    def _():
                        issue(g + 4, b)
            return z

        lax.fori_loop(0, (nbp + 3) // 4, quads, 0)

        pcoef = pbv[pl.ds(0, LN)][0]

        def fin(r, z):
            for c in range(HID // LN):
                bc = pbv[pl.ds(LN + c * LN, LN)]
                v = acc[r, pl.ds(c * LN, LN)] + bc
                v = jnp.where(v >= 0.0, v, pcoef * v)
                acc[r, pl.ds(c * LN, LN)] = v
            return z
        lax.fori_loop(0, R, fin, 0)
        pltpu.sync_copy(acc, hout.at[pl.ds(lo, R), :])

    pl.run_scoped(
        phase_b,
        pltpu.VMEM((R, HID), jnp.float32),
        pltpu.VMEM((GK, HID), jnp.int32),
        pltpu.VMEM((GK, HID), jnp.int32),
        pltpu.VMEM((GK, HID), jnp.int32),
        pltpu.VMEM((GK, HID), jnp.int32),
    )


def _make_sc_layer(row0, row1, build):
    mesh = plsc.VectorSubcoreMesh(core_axis_name="c", subcore_axis_name="s",
                                  num_cores=NC, num_subcores=NS)
    if build:
        out_type = [
            jax.ShapeDtypeStruct((NP, HID), jnp.float32),
            jax.ShapeDtypeStruct((NW * CAP,), jnp.int32),
            jax.ShapeDtypeStruct((NW * CAP,), jnp.int32),
            jax.ShapeDtypeStruct((NW * CAP,), jnp.int32),
            jax.ShapeDtypeStruct((NW * LN,), jnp.int32),
        ]
    else:
        out_type = jax.ShapeDtypeStruct((NP, HID), jnp.float32)
    return pl.kernel(
        functools.partial(_sc_body, row0, row1, build),
        out_type=out_type,
        mesh=mesh,
        compiler_params=pltpu.CompilerParams(needs_layout_passes=False),
        scratch_types=[
            pltpu.VMEM((CAP,), jnp.int32),
            pltpu.VMEM((CAP,), jnp.int32),
            pltpu.VMEM((CAP,), jnp.float32),
            pltpu.VMEM((CAP,), jnp.float32),
            pltpu.VMEM((R,), jnp.float32),
            pltpu.VMEM((R,), jnp.float32),
            pltpu.VMEM((272,), jnp.float32),
            pltpu.SMEM((1,), jnp.int32),
            pltpu.SemaphoreType.DMA,
            pltpu.SemaphoreType.DMA,
            pltpu.SemaphoreType.DMA,
            pltpu.SemaphoreType.DMA,
        ],
    )


def _fold(W, a):
    C = a.shape[1]
    cols = [W[:, h * C:(h + 1) * C] @ a[h] for h in range(a.shape[0])]
    return jnp.stack(cols, axis=0)


def kernel(x, edge_index, edge_attr, batch, W1, We1, as1, ad1, ae1, b1, p1,
           W2, We2, as2, ad2, ae2, b2, p2, G1, gb1, gp1, G2, gb2, gp2, G3, gb3):
    xpad = jnp.pad(x, ((0, NP - NREAL), (0, 0)))
    esrc = jnp.pad(edge_index[0], (0, EP - E))
    edst = jnp.pad(edge_index[1], (0, EP - E), constant_values=-1)
    ea_pad = jnp.pad(edge_attr, ((0, EP - E), (0, 0)))

    def _spread(rows, K):
        out = jnp.zeros((32, K))
        for i, r in enumerate(rows):
            out = out.at[8 * i].set(r)
        return out

    f1s, f1d = _fold(W1, as1), _fold(W1, ad1)
    f2s, f2d = _fold(W2, as2), _fold(W2, ad2)
    fe1, fe2 = _fold(We1, ae1), _fold(We2, ae2)
    wsd1T = _spread([f1s[0], f1s[1], f1d[0], f1d[1]], CLIP)
    wsd2T = _spread([f2s[0], f2s[1], f2d[0], f2d[1]], HID)
    weT = _spread([fe1[0], fe1[1], fe2[0], fe2[1]], CLIP)
    pb1 = jnp.concatenate([p1[None], jnp.zeros(15), b1])
    pb2 = jnp.concatenate([p2[None], jnp.zeros(15), b2])

    perm = jnp.arange(H * HID).reshape(16, 2, 16).transpose(0, 2, 1).reshape(-1)
    W1p = W1[:, perm]
    W2p = W2[:, perm]

    aleT = _edge_logits(ea_pad, weT)
    xp1, asd1T = _proj(xpad, W1p, wsd1T)
    aleT_f = aleT.reshape(-1)
    xp1u = lax.bitcast_convert_type(xp1.reshape(NP, HID, 2), jnp.int32)
    h1, l_src, l_dl, l_eid, cnts = _make_sc_layer(0, 8, True)(
        esrc, edst, aleT_f, asd1T.reshape(-1), xp1u, pb1)
    xp2, asd2T = _proj(h1, W2p, wsd2T)
    xp2u = lax.bitcast_convert_type(xp2.reshape(NP, HID, 2), jnp.int32)
    h2 = _make_sc_layer(16, 24, False)(
        aleT_f, asd2T.reshape(-1), xp2u, pb2, l_src, l_dl, l_eid, cnts)

    g = _mlp(h2, G1, gb1, gp1, G2, gb2, gp2, G3, gb3)
    g1 = g[:NREAL, 0].reshape(10, 1, 1000)
    b3 = batch.reshape(10, 1, 1000)
    h2r = h2[:NREAL]
    pooled = _pool(g1, b3, h2r)
    return pooled.reshape(B, PRE, MB)
